# Initial kernel scaffold; baseline (speedup 1.0000x reference)
import jax
import jax.numpy as jnp
from jax.experimental import pallas as pl


def kernel(x):
    raise NotImplementedError("write your pallas kernel here")


# speedup vs baseline: 159.8444x; 159.8444x over previous
import dataclasses
import functools

import jax
import jax.numpy as jnp
from jax import lax
from jax.experimental import pallas as pl
from jax.experimental.pallas import tpu as pltpu
from jax.experimental.pallas import tpu_sc as plsc

_L = 16
_NSUB = 16
_NCORE = 2
_NPIX = 512 * 512
_PPT = _NPIX // _NSUB
_NVEC = _PPT // _L
_NBINS = 256
_NCHUNK = _NBINS // _L
_NCH = 48
_CPC = _NCH // _NCORE


def _he_kernel(x_hbm, o_hbm, in_v, idx_v, out_v, h2d_v, hist_v, hall_v,
               lut_v, shared):
    cid = lax.axis_index("c")
    sid = lax.axis_index("s")
    base = sid * _PPT
    iota_i = lax.iota(jnp.int32, _L)
    iota_f = iota_i.astype(jnp.float32)
    ones = jnp.full((_L,), 1.0, jnp.float32)
    zeros = jnp.full((_L,), 0.0, jnp.float32)

    @pl.loop(0, _CPC)
    def _channel(j):
        ch = cid * _CPC + j
        pltpu.sync_copy(x_hbm.at[ch, pl.ds(base, _PPT)], in_v)

        for r in range(_NSUB):
            for k in range(_NCHUNK):
                h2d_v[r, pl.ds(k * _L, _L)] = zeros

        @pl.loop(0, _NVEC)
        def _hist(i):
            v = in_v[pl.ds(i * _L, _L)]
            xi = (v * 255.0).astype(jnp.int32)
            idx_v[pl.ds(i * _L, _L)] = xi
            plsc.addupdate_scatter(h2d_v, [iota_i, xi], ones)

        for k in range(_NCHUNK):
            acc = h2d_v[0, pl.ds(k * _L, _L)]
            for r in range(1, _NSUB):
                acc = acc + h2d_v[r, pl.ds(k * _L, _L)]
            hist_v[pl.ds(k * _L, _L)] = acc

        pltpu.sync_copy(hist_v, shared.at[sid])
        plsc.subcore_barrier()
        pltpu.sync_copy(shared, hall_v)
        plsc.subcore_barrier()
        for k in range(_NCHUNK):
            acc = hall_v[0, pl.ds(k * _L, _L)]
            for r in range(1, _NSUB):
                acc = acc + hall_v[r, pl.ds(k * _L, _L)]
            hist_v[pl.ds(k * _L, _L)] = acc

        last_val = jnp.float32(0.0)
        for k in range(_NCHUNK):
            h = hist_v[pl.ds(k * _L, _L)]
            m = jnp.max(jnp.where(h != 0.0, iota_i, -1))
            chunk_last = jnp.sum(jnp.where(iota_i == m, h, 0.0))
            last_val = jnp.where(m >= 0, chunk_last, last_val)

        def _floor_nonneg(v):
            return v.astype(jnp.int32).astype(jnp.float32)

        last_vec = jnp.broadcast_to(last_val, (_L,))
        step = _floor_nonneg((jnp.float32(_NPIX) - last_vec) / 255.0)
        safe_step = jnp.maximum(step, 1.0)
        half = _floor_nonneg(step * 0.5)
        is_id = step == 0.0

        carry = jnp.float32(0.0)
        for k in range(_NCHUNK):
            h = hist_v[pl.ds(k * _L, _L)]
            cs = jnp.cumsum(h)
            excl = cs - h + carry
            carry = carry + jnp.sum(h)
            lv = _floor_nonneg((excl + half) / safe_step)
            lv = jnp.clip(lv, 0.0, 255.0)
            lv = jnp.where(is_id, iota_f + float(k * _L), lv)
            lut_v[pl.ds(k * _L, _L)] = lv / 255.0

        @pl.loop(0, _NVEC)
        def _gather(i):
            xi = idx_v[pl.ds(i * _L, _L)]
            out_v[pl.ds(i * _L, _L)] = plsc.load_gather(lut_v, [xi])

        pltpu.sync_copy(out_v, o_hbm.at[ch, pl.ds(base, _PPT)])


@jax.jit
def kernel(x):
    xf = x.reshape(_NCH, _NPIX)
    cp = pltpu.CompilerParams()
    if "needs_layout_passes" in pltpu.CompilerParams.__dataclass_fields__:
        cp = dataclasses.replace(cp, needs_layout_passes=False)
    run = pl.kernel(
        _he_kernel,
        out_type=jax.ShapeDtypeStruct((_NCH, _NPIX), jnp.float32),
        mesh=plsc.VectorSubcoreMesh(core_axis_name="c", subcore_axis_name="s"),
        scratch_types=[
            pltpu.VMEM((_PPT,), jnp.float32),
            pltpu.VMEM((_PPT,), jnp.int32),
            pltpu.VMEM((_PPT,), jnp.float32),
            pltpu.VMEM((_NSUB, _NBINS), jnp.float32),
            pltpu.VMEM((_NBINS,), jnp.float32),
            pltpu.VMEM((_NSUB, _NBINS), jnp.float32),
            pltpu.VMEM((_NBINS,), jnp.float32),
            pltpu.VMEM_SHARED((_NSUB, _NBINS), jnp.float32),
        ],
        compiler_params=cp,
    )
    return run(xf).reshape(x.shape)


# speedup vs baseline: 181.9912x; 1.1386x over previous
import dataclasses
import functools

import jax
import jax.numpy as jnp
from jax import lax
from jax.experimental import pallas as pl
from jax.experimental.pallas import tpu as pltpu
from jax.experimental.pallas import tpu_sc as plsc

_L = 16
_NSUB = 16
_NCORE = 2
_NPIX = 512 * 512
_PPT = _NPIX // _NSUB
_NVEC = _PPT // _L
_NBINS = 256
_NCHUNK = _NBINS // _L
_NCH = 48
_CPC = _NCH // _NCORE
_U = 8


def _he_kernel(x_hbm, o_hbm, in0, in1, out0, out1, h2d_v, hist_v, hall_v,
               lut_v, shared, sem_in, sem_out):
    cid = lax.axis_index("c")
    sid = lax.axis_index("s")
    base = sid * _PPT
    ch0 = cid * _CPC
    iota_i = lax.iota(jnp.int32, _L)
    iota_f = iota_i.astype(jnp.float32)
    ones = jnp.full((_L,), 1.0, jnp.float32)
    zeros = jnp.full((_L,), 0.0, jnp.float32)
    ins = (in0, in1)
    outs = (out0, out1)

    pltpu.async_copy(x_hbm.at[ch0, pl.ds(base, _PPT)], in0, sem_in)

    def _one_channel(jl, b):
        ch = ch0 + jl
        in_b = ins[b]
        out_b = outs[b]

        pltpu.make_async_copy(
            x_hbm.at[ch, pl.ds(base, _PPT)], in_b, sem_in).wait()

        @pl.when(jl + 1 < _CPC)
        def _():
            pltpu.async_copy(
                x_hbm.at[ch + 1, pl.ds(base, _PPT)], ins[1 - b], sem_in)

        for r in range(_NSUB):
            for k in range(_NCHUNK):
                h2d_v[r, pl.ds(k * _L, _L)] = zeros

        @pl.loop(0, _NVEC, step=_U)
        def _hist(i):
            for u in range(_U):
                v = in_b[pl.ds((i + u) * _L, _L)]
                xi = (v * 255.0).astype(jnp.int32)
                plsc.addupdate_scatter(h2d_v, [iota_i, xi], ones)

        for k in range(_NCHUNK):
            acc = h2d_v[0, pl.ds(k * _L, _L)]
            for r in range(1, _NSUB):
                acc = acc + h2d_v[r, pl.ds(k * _L, _L)]
            hist_v[pl.ds(k * _L, _L)] = acc

        pltpu.sync_copy(hist_v, shared.at[sid])
        plsc.subcore_barrier()
        pltpu.sync_copy(shared, hall_v)
        plsc.subcore_barrier()
        for k in range(_NCHUNK):
            acc = hall_v[0, pl.ds(k * _L, _L)]
            for r in range(1, _NSUB):
                acc = acc + hall_v[r, pl.ds(k * _L, _L)]
            hist_v[pl.ds(k * _L, _L)] = acc

        last_val = jnp.float32(0.0)
        for k in range(_NCHUNK):
            h = hist_v[pl.ds(k * _L, _L)]
            m = jnp.max(jnp.where(h != 0.0, iota_i, -1))
            chunk_last = jnp.sum(jnp.where(iota_i == m, h, 0.0))
            last_val = jnp.where(m >= 0, chunk_last, last_val)

        def _floor_nonneg(v):
            return v.astype(jnp.int32).astype(jnp.float32)

        last_vec = jnp.broadcast_to(last_val, (_L,))
        step = _floor_nonneg((jnp.float32(_NPIX) - last_vec) / 255.0)
        safe_step = jnp.maximum(step, 1.0)
        half = _floor_nonneg(step * 0.5)
        is_id = step == 0.0

        carry = jnp.float32(0.0)
        for k in range(_NCHUNK):
            h = hist_v[pl.ds(k * _L, _L)]
            cs = jnp.cumsum(h)
            excl = cs - h + carry
            carry = carry + jnp.sum(h)
            lv = _floor_nonneg((excl + half) / safe_step)
            lv = jnp.clip(lv, 0.0, 255.0)
            lv = jnp.where(is_id, iota_f + float(k * _L), lv)
            lut_v[pl.ds(k * _L, _L)] = lv / 255.0

        @pl.when(jl >= 2)
        def _():
            pltpu.make_async_copy(
                out_b, o_hbm.at[ch, pl.ds(base, _PPT)], sem_out).wait()

        @pl.loop(0, _NVEC, step=_U)
        def _gather(i):
            for u in range(_U):
                v = in_b[pl.ds((i + u) * _L, _L)]
                xi = (v * 255.0).astype(jnp.int32)
                out_b[pl.ds((i + u) * _L, _L)] = plsc.load_gather(lut_v, [xi])

        pltpu.async_copy(out_b, o_hbm.at[ch, pl.ds(base, _PPT)], sem_out)

    @pl.loop(0, _CPC, step=2)
    def _channels(j):
        _one_channel(j, 0)
        _one_channel(j + 1, 1)

    for b in range(2):
        pltpu.make_async_copy(
            outs[b], o_hbm.at[ch0 + _CPC - 2 + b, pl.ds(base, _PPT)],
            sem_out).wait()


@jax.jit
def kernel(x):
    xf = x.reshape(_NCH, _NPIX)
    cp = pltpu.CompilerParams()
    if "needs_layout_passes" in pltpu.CompilerParams.__dataclass_fields__:
        cp = dataclasses.replace(cp, needs_layout_passes=False)
    run = pl.kernel(
        _he_kernel,
        out_type=jax.ShapeDtypeStruct((_NCH, _NPIX), jnp.float32),
        mesh=plsc.VectorSubcoreMesh(core_axis_name="c", subcore_axis_name="s"),
        scratch_types=[
            pltpu.VMEM((_PPT,), jnp.float32),
            pltpu.VMEM((_PPT,), jnp.float32),
            pltpu.VMEM((_PPT,), jnp.float32),
            pltpu.VMEM((_PPT,), jnp.float32),
            pltpu.VMEM((_NSUB, _NBINS), jnp.float32),
            pltpu.VMEM((_NBINS,), jnp.float32),
            pltpu.VMEM((_NSUB, _NBINS), jnp.float32),
            pltpu.VMEM((_NBINS,), jnp.float32),
            pltpu.VMEM_SHARED((_NSUB, _NBINS), jnp.float32),
            pltpu.SemaphoreType.DMA,
            pltpu.SemaphoreType.DMA,
        ],
        compiler_params=cp,
    )
    return run(xf).reshape(x.shape)


# speedup vs baseline: 375.0922x; 2.0610x over previous
import dataclasses
import functools

import jax
import jax.numpy as jnp
from jax import lax
from jax.experimental import pallas as pl
from jax.experimental.pallas import tpu as pltpu
from jax.experimental.pallas import tpu_sc as plsc

_L = 16
_NSUB = 16
_NCORE = 2
_NPIX = 512 * 512
_PPT = _NPIX // _NSUB
_NVEC = _PPT // _L
_NBINS = 256
_NCHUNK = _NBINS // _L
_NCH = 48
_CPC = _NCH // _NCORE
_U = 8


def _he_kernel(x_hbm, o_hbm, in0, in1, out0, out1, h2d_v, hist_v, hall_v,
               lut_v, shared, sem_in, sem_out):
    cid = lax.axis_index("c")
    sid = lax.axis_index("s")
    base = sid * _PPT
    ch0 = cid * _CPC
    iota_i = lax.iota(jnp.int32, _L)
    iota_f = iota_i.astype(jnp.float32)
    ones = jnp.full((_L,), 1.0, jnp.float32)
    zeros = jnp.full((_L,), 0.0, jnp.float32)
    rowoff = iota_i * _NBINS
    ins = (in0, in1)
    outs = (out0, out1)

    pltpu.async_copy(x_hbm.at[ch0, pl.ds(base, _PPT)], in0, sem_in)

    def _one_channel(jl, b):
        ch = ch0 + jl
        in_b = ins[b]
        out_b = outs[b]

        pltpu.make_async_copy(
            x_hbm.at[ch, pl.ds(base, _PPT)], in_b, sem_in).wait()

        @pl.when(jl + 1 < _CPC)
        def _():
            pltpu.async_copy(
                x_hbm.at[ch + 1, pl.ds(base, _PPT)], ins[1 - b], sem_in)

        for r in range(_NSUB * _NCHUNK):
            h2d_v[pl.ds(r * _L, _L)] = zeros

        @pl.loop(0, _NVEC, step=_U)
        def _hist(i):
            idxs = []
            for u in range(_U):
                v = in_b[pl.ds((i + u) * _L, _L)]
                xi = (v * 255.0).astype(jnp.int32)
                idxs.append(rowoff + xi)
            for u in range(_U):
                plsc.addupdate_scatter(h2d_v, [idxs[u]], ones)

        for k in range(_NCHUNK):
            acc = h2d_v[pl.ds(k * _L, _L)]
            for r in range(1, _NSUB):
                acc = acc + h2d_v[pl.ds(r * _NBINS + k * _L, _L)]
            hist_v[pl.ds(k * _L, _L)] = acc

        pltpu.sync_copy(hist_v, shared.at[sid])
        plsc.subcore_barrier()
        pltpu.sync_copy(shared, hall_v)
        plsc.subcore_barrier()
        for k in range(_NCHUNK):
            acc = hall_v[0, pl.ds(k * _L, _L)]
            for r in range(1, _NSUB):
                acc = acc + hall_v[r, pl.ds(k * _L, _L)]
            hist_v[pl.ds(k * _L, _L)] = acc

        last_val = jnp.float32(0.0)
        for k in range(_NCHUNK):
            h = hist_v[pl.ds(k * _L, _L)]
            m = jnp.max(jnp.where(h != 0.0, iota_i, -1))
            chunk_last = jnp.sum(jnp.where(iota_i == m, h, 0.0))
            last_val = jnp.where(m >= 0, chunk_last, last_val)

        def _floor_nonneg(v):
            return v.astype(jnp.int32).astype(jnp.float32)

        last_vec = jnp.broadcast_to(last_val, (_L,))
        step = _floor_nonneg((jnp.float32(_NPIX) - last_vec) / 255.0)
        safe_step = jnp.maximum(step, 1.0)
        half = _floor_nonneg(step * 0.5)
        is_id = step == 0.0

        carry = jnp.float32(0.0)
        for k in range(_NCHUNK):
            h = hist_v[pl.ds(k * _L, _L)]
            cs = jnp.cumsum(h)
            excl = cs - h + carry
            carry = carry + jnp.sum(h)
            lv = _floor_nonneg((excl + half) / safe_step)
            lv = jnp.clip(lv, 0.0, 255.0)
            lv = jnp.where(is_id, iota_f + float(k * _L), lv)
            lut_v[pl.ds(k * _L, _L)] = lv / 255.0

        @pl.when(jl >= 2)
        def _():
            pltpu.make_async_copy(
                out_b, o_hbm.at[ch, pl.ds(base, _PPT)], sem_out).wait()

        @pl.loop(0, _NVEC, step=_U)
        def _gather(i):
            xis = []
            for u in range(_U):
                v = in_b[pl.ds((i + u) * _L, _L)]
                xis.append((v * 255.0).astype(jnp.int32))
            outs_u = [plsc.load_gather(lut_v, [xi]) for xi in xis]
            for u in range(_U):
                out_b[pl.ds((i + u) * _L, _L)] = outs_u[u]

        pltpu.async_copy(out_b, o_hbm.at[ch, pl.ds(base, _PPT)], sem_out)

    @pl.loop(0, _CPC, step=2)
    def _channels(j):
        _one_channel(j, 0)
        _one_channel(j + 1, 1)

    for b in range(2):
        pltpu.make_async_copy(
            outs[b], o_hbm.at[ch0 + _CPC - 2 + b, pl.ds(base, _PPT)],
            sem_out).wait()


@jax.jit
def kernel(x):
    xf = x.reshape(_NCH, _NPIX)
    cp = pltpu.CompilerParams()
    if "needs_layout_passes" in pltpu.CompilerParams.__dataclass_fields__:
        cp = dataclasses.replace(cp, needs_layout_passes=False)
    run = pl.kernel(
        _he_kernel,
        out_type=jax.ShapeDtypeStruct((_NCH, _NPIX), jnp.float32),
        mesh=plsc.VectorSubcoreMesh(core_axis_name="c", subcore_axis_name="s"),
        scratch_types=[
            pltpu.VMEM((_PPT,), jnp.float32),
            pltpu.VMEM((_PPT,), jnp.float32),
            pltpu.VMEM((_PPT,), jnp.float32),
            pltpu.VMEM((_PPT,), jnp.float32),
            pltpu.VMEM((_NSUB * _NBINS,), jnp.float32),
            pltpu.VMEM((_NBINS,), jnp.float32),
            pltpu.VMEM((_NSUB, _NBINS), jnp.float32),
            pltpu.VMEM((_NBINS,), jnp.float32),
            pltpu.VMEM_SHARED((_NSUB, _NBINS), jnp.float32),
            pltpu.SemaphoreType.DMA,
            pltpu.SemaphoreType.DMA,
        ],
        compiler_params=cp,
    )
    return run(xf).reshape(x.shape)


# speedup vs baseline: 409.3076x; 1.0912x over previous
import dataclasses
import functools

import jax
import jax.numpy as jnp
from jax import lax
from jax.experimental import pallas as pl
from jax.experimental.pallas import tpu as pltpu
from jax.experimental.pallas import tpu_sc as plsc

_L = 16
_NSUB = 16
_NCORE = 2
_NPIX = 512 * 512
_PPT = _NPIX // _NSUB
_NVEC = _PPT // _L
_NBINS = 256
_NCHUNK = _NBINS // _L
_NCH = 48
_CPC = _NCH // _NCORE
_U = 16


def _he_kernel(x_hbm, o_hbm, in0, in1, out0, out1, h2d_v, hist_v, hall_v,
               lut_v, shared, sem_in, sem_out):
    cid = lax.axis_index("c")
    sid = lax.axis_index("s")
    base = sid * _PPT
    ch0 = cid * _CPC
    iota_i = lax.iota(jnp.int32, _L)
    iota_f = iota_i.astype(jnp.float32)
    ones = jnp.full((_L,), 1.0, jnp.float32)
    zeros = jnp.full((_L,), 0.0, jnp.float32)
    rowoff = iota_i * _NBINS
    ins = (in0, in1)
    outs = (out0, out1)

    pltpu.async_copy(x_hbm.at[ch0, pl.ds(base, _PPT)], in0, sem_in)

    def _one_channel(jl, b):
        ch = ch0 + jl
        in_b = ins[b]
        out_b = outs[b]

        pltpu.make_async_copy(
            x_hbm.at[ch, pl.ds(base, _PPT)], in_b, sem_in).wait()

        @pl.when(jl + 1 < _CPC)
        def _():
            pltpu.async_copy(
                x_hbm.at[ch + 1, pl.ds(base, _PPT)], ins[1 - b], sem_in)

        for r in range(_NSUB * _NCHUNK):
            h2d_v[pl.ds(r * _L, _L)] = zeros

        @pl.loop(0, _NVEC, step=_U)
        def _hist(i):
            idxs = []
            for u in range(_U):
                v = in_b[pl.ds((i + u) * _L, _L)]
                xi = (v * 255.0).astype(jnp.int32)
                idxs.append(rowoff + xi)
            for u in range(_U):
                plsc.addupdate_scatter(h2d_v, [idxs[u]], ones)

        for k in range(_NCHUNK):
            acc = h2d_v[pl.ds(k * _L, _L)]
            for r in range(1, _NSUB):
                acc = acc + h2d_v[pl.ds(r * _NBINS + k * _L, _L)]
            hist_v[pl.ds(k * _L, _L)] = acc

        pltpu.sync_copy(hist_v, shared.at[sid])
        plsc.subcore_barrier()
        pltpu.sync_copy(shared, hall_v)
        plsc.subcore_barrier()
        for k in range(_NCHUNK):
            acc = hall_v[0, pl.ds(k * _L, _L)]
            for r in range(1, _NSUB):
                acc = acc + hall_v[r, pl.ds(k * _L, _L)]
            hist_v[pl.ds(k * _L, _L)] = acc

        last_val = jnp.float32(0.0)
        for k in range(_NCHUNK):
            h = hist_v[pl.ds(k * _L, _L)]
            m = jnp.max(jnp.where(h != 0.0, iota_i, -1))
            chunk_last = jnp.sum(jnp.where(iota_i == m, h, 0.0))
            last_val = jnp.where(m >= 0, chunk_last, last_val)

        def _floor_nonneg(v):
            return v.astype(jnp.int32).astype(jnp.float32)

        last_vec = jnp.broadcast_to(last_val, (_L,))
        step = _floor_nonneg((jnp.float32(_NPIX) - last_vec) / 255.0)
        safe_step = jnp.maximum(step, 1.0)
        half = _floor_nonneg(step * 0.5)
        is_id = step == 0.0

        carry = jnp.float32(0.0)
        for k in range(_NCHUNK):
            h = hist_v[pl.ds(k * _L, _L)]
            cs = jnp.cumsum(h)
            excl = cs - h + carry
            carry = carry + jnp.sum(h)
            lv = _floor_nonneg((excl + half) / safe_step)
            lv = jnp.clip(lv, 0.0, 255.0)
            lv = jnp.where(is_id, iota_f + float(k * _L), lv)
            lut_v[pl.ds(k * _L, _L)] = lv / 255.0

        @pl.when(jl >= 2)
        def _():
            pltpu.make_async_copy(
                out_b, o_hbm.at[ch, pl.ds(base, _PPT)], sem_out).wait()

        @pl.loop(0, _NVEC, step=_U)
        def _gather(i):
            xis = []
            for u in range(_U):
                v = in_b[pl.ds((i + u) * _L, _L)]
                xis.append((v * 255.0).astype(jnp.int32))
            outs_u = [plsc.load_gather(lut_v, [xi]) for xi in xis]
            for u in range(_U):
                out_b[pl.ds((i + u) * _L, _L)] = outs_u[u]

        pltpu.async_copy(out_b, o_hbm.at[ch, pl.ds(base, _PPT)], sem_out)

    @pl.loop(0, _CPC, step=2)
    def _channels(j):
        _one_channel(j, 0)
        _one_channel(j + 1, 1)

    for b in range(2):
        pltpu.make_async_copy(
            outs[b], o_hbm.at[ch0 + _CPC - 2 + b, pl.ds(base, _PPT)],
            sem_out).wait()


@jax.jit
def kernel(x):
    xf = x.reshape(_NCH, _NPIX)
    cp = pltpu.CompilerParams()
    if "needs_layout_passes" in pltpu.CompilerParams.__dataclass_fields__:
        cp = dataclasses.replace(cp, needs_layout_passes=False)
    run = pl.kernel(
        _he_kernel,
        out_type=jax.ShapeDtypeStruct((_NCH, _NPIX), jnp.float32),
        mesh=plsc.VectorSubcoreMesh(core_axis_name="c", subcore_axis_name="s"),
        scratch_types=[
            pltpu.VMEM((_PPT,), jnp.float32),
            pltpu.VMEM((_PPT,), jnp.float32),
            pltpu.VMEM((_PPT,), jnp.float32),
            pltpu.VMEM((_PPT,), jnp.float32),
            pltpu.VMEM((_NSUB * _NBINS,), jnp.float32),
            pltpu.VMEM((_NBINS,), jnp.float32),
            pltpu.VMEM((_NSUB, _NBINS), jnp.float32),
            pltpu.VMEM((_NBINS,), jnp.float32),
            pltpu.VMEM_SHARED((_NSUB, _NBINS), jnp.float32),
            pltpu.SemaphoreType.DMA,
            pltpu.SemaphoreType.DMA,
        ],
        compiler_params=cp,
    )
    return run(xf).reshape(x.shape)


# speedup vs baseline: 668.9507x; 1.6343x over previous
import dataclasses
import functools

import jax
import jax.numpy as jnp
from jax import lax
from jax.experimental import pallas as pl
from jax.experimental.pallas import tpu as pltpu
from jax.experimental.pallas import tpu_sc as plsc

_L = 16
_NSUB = 16
_NCORE = 2
_H = 512
_W = 512
_RPT = _H // _NSUB
_NVROW = _W // _L
_NBINS = 256
_NCHUNK = _NBINS // _L
_NCH = 48
_CPC = _NCH // _NCORE
_U = 16


def _he_kernel(x_hbm, o_hbm, in0, in1, out0, out1, h2d_v, hist_v, hall_v,
               lut_v, shared, sem_in, sem_out):
    cid = lax.axis_index("c")
    sid = lax.axis_index("s")
    row0 = sid * _RPT
    ch0 = cid * _CPC
    iota_i = lax.iota(jnp.int32, _L)
    iota_f = iota_i.astype(jnp.float32)
    ones = jnp.full((_L,), 1.0, jnp.float32)
    zeros = jnp.full((_L,), 0.0, jnp.float32)
    rowoff = iota_i * _NBINS
    ins = (in0, in1)
    outs = (out0, out1)

    pltpu.async_copy(x_hbm.at[ch0, pl.ds(row0, _RPT), :], in0, sem_in)

    def _one_channel(jl, b):
        ch = ch0 + jl
        in_b = ins[b]
        out_b = outs[b]

        pltpu.make_async_copy(
            x_hbm.at[ch, pl.ds(row0, _RPT), :], in_b, sem_in).wait()

        @pl.when(jl + 1 < _CPC)
        def _():
            pltpu.async_copy(
                x_hbm.at[ch + 1, pl.ds(row0, _RPT), :], ins[1 - b], sem_in)

        for r in range(_NSUB * _NCHUNK):
            h2d_v[pl.ds(r * _L, _L)] = zeros

        @pl.loop(0, _RPT)
        def _hist(r):
            for k0 in range(0, _NVROW, _U):
                idxs = []
                for k in range(k0, k0 + _U):
                    v = in_b[r, pl.ds(k * _L, _L)]
                    xi = (v * 255.0).astype(jnp.int32)
                    idxs.append(rowoff + xi)
                for idx in idxs:
                    plsc.addupdate_scatter(h2d_v, [idx], ones)

        for k in range(_NCHUNK):
            acc = h2d_v[pl.ds(k * _L, _L)]
            for r in range(1, _NSUB):
                acc = acc + h2d_v[pl.ds(r * _NBINS + k * _L, _L)]
            hist_v[pl.ds(k * _L, _L)] = acc

        pltpu.sync_copy(hist_v, shared.at[sid])
        plsc.subcore_barrier()
        pltpu.sync_copy(shared, hall_v)
        plsc.subcore_barrier()
        for k in range(_NCHUNK):
            acc = hall_v[0, pl.ds(k * _L, _L)]
            for r in range(1, _NSUB):
                acc = acc + hall_v[r, pl.ds(k * _L, _L)]
            hist_v[pl.ds(k * _L, _L)] = acc

        last_val = jnp.float32(0.0)
        for k in range(_NCHUNK):
            h = hist_v[pl.ds(k * _L, _L)]
            m = jnp.max(jnp.where(h != 0.0, iota_i, -1))
            chunk_last = jnp.sum(jnp.where(iota_i == m, h, 0.0))
            last_val = jnp.where(m >= 0, chunk_last, last_val)

        def _floor_nonneg(v):
            return v.astype(jnp.int32).astype(jnp.float32)

        last_vec = jnp.broadcast_to(last_val, (_L,))
        step = _floor_nonneg((jnp.float32(_H * _W) - last_vec) / 255.0)
        safe_step = jnp.maximum(step, 1.0)
        half = _floor_nonneg(step * 0.5)
        is_id = step == 0.0

        carry = jnp.float32(0.0)
        for k in range(_NCHUNK):
            h = hist_v[pl.ds(k * _L, _L)]
            cs = jnp.cumsum(h)
            excl = cs - h + carry
            carry = carry + jnp.sum(h)
            lv = _floor_nonneg((excl + half) / safe_step)
            lv = jnp.clip(lv, 0.0, 255.0)
            lv = jnp.where(is_id, iota_f + float(k * _L), lv)
            lut_v[pl.ds(k * _L, _L)] = lv / 255.0

        @pl.when(jl >= 2)
        def _():
            pltpu.make_async_copy(
                out_b, o_hbm.at[ch, pl.ds(row0, _RPT), :], sem_out).wait()

        @pl.loop(0, _RPT)
        def _gather(r):
            for k0 in range(0, _NVROW, _U):
                xis = []
                for k in range(k0, k0 + _U):
                    v = in_b[r, pl.ds(k * _L, _L)]
                    xis.append((v * 255.0).astype(jnp.int32))
                outs_u = [plsc.load_gather(lut_v, [xi]) for xi in xis]
                for k in range(k0, k0 + _U):
                    out_b[r, pl.ds(k * _L, _L)] = outs_u[k - k0]

        pltpu.async_copy(out_b, o_hbm.at[ch, pl.ds(row0, _RPT), :], sem_out)

    @pl.loop(0, _CPC, step=2)
    def _channels(j):
        _one_channel(j, 0)
        _one_channel(j + 1, 1)

    for b in range(2):
        pltpu.make_async_copy(
            outs[b], o_hbm.at[ch0 + _CPC - 2 + b, pl.ds(row0, _RPT), :],
            sem_out).wait()


@jax.jit
def kernel(x):
    xf = x.reshape(_NCH, _H, _W)
    cp = pltpu.CompilerParams(use_tc_tiling_on_sc=True)
    if "needs_layout_passes" in pltpu.CompilerParams.__dataclass_fields__:
        cp = dataclasses.replace(cp, needs_layout_passes=False)
    run = pl.kernel(
        _he_kernel,
        out_type=jax.ShapeDtypeStruct((_NCH, _H, _W), jnp.float32),
        mesh=plsc.VectorSubcoreMesh(core_axis_name="c", subcore_axis_name="s"),
        scratch_types=[
            pltpu.VMEM((_RPT, _W), jnp.float32),
            pltpu.VMEM((_RPT, _W), jnp.float32),
            pltpu.VMEM((_RPT, _W), jnp.float32),
            pltpu.VMEM((_RPT, _W), jnp.float32),
            pltpu.VMEM((_NSUB * _NBINS,), jnp.float32),
            pltpu.VMEM((_NBINS,), jnp.float32),
            pltpu.VMEM((_NSUB, _NBINS), jnp.float32),
            pltpu.VMEM((_NBINS,), jnp.float32),
            pltpu.VMEM_SHARED((_NSUB, _NBINS), jnp.float32),
            pltpu.SemaphoreType.DMA,
            pltpu.SemaphoreType.DMA,
        ],
        compiler_params=cp,
    )
    return run(xf).reshape(x.shape)


# speedup vs baseline: 704.5832x; 1.0533x over previous
import dataclasses
import functools

import jax
import jax.numpy as jnp
from jax import lax
from jax.experimental import pallas as pl
from jax.experimental.pallas import tpu as pltpu
from jax.experimental.pallas import tpu_sc as plsc

_L = 16
_NSUB = 16
_NCORE = 2
_H = 512
_W = 512
_RPT = _H // _NSUB
_NVROW = _W // _L
_NBINS = 256
_NCHUNK = _NBINS // _L
_NCH = 48
_CPC = _NCH // _NCORE
_U = 16


def _he_kernel(x_hbm, o_hbm, in0, in1, out0, out1, h2d_v, hist_v, hall_v,
               lut_v, cs_buf, shared, sem_in, sem_out, sem_pub):
    cid = lax.axis_index("c")
    sid = lax.axis_index("s")
    row0 = sid * _RPT
    ch0 = cid * _CPC
    iota_i = lax.iota(jnp.int32, _L)
    iota_f = iota_i.astype(jnp.float32)
    ones = jnp.full((_L,), 1.0, jnp.float32)
    zeros = jnp.full((_L,), 0.0, jnp.float32)
    rowoff = iota_i * _NBINS
    ins = (in0, in1)
    outs = (out0, out1)

    cs_buf[pl.ds(0, _L)] = zeros

    pltpu.async_copy(x_hbm.at[ch0, pl.ds(row0, _RPT), :], in0, sem_in)

    def _one_channel(jl, b):
        ch = ch0 + jl
        in_b = ins[b]
        out_b = outs[b]

        pltpu.make_async_copy(
            x_hbm.at[ch, pl.ds(row0, _RPT), :], in_b, sem_in).wait()

        @pl.when(jl + 1 < _CPC)
        def _():
            pltpu.async_copy(
                x_hbm.at[ch + 1, pl.ds(row0, _RPT), :], ins[1 - b], sem_in)

        @pl.loop(0, _RPT)
        def _hist(r):
            for k0 in range(0, _NVROW, _U):
                idxs = []
                for k in range(k0, k0 + _U):
                    v = in_b[r, pl.ds(k * _L, _L)]
                    xi = (v * 255.0).astype(jnp.int32)
                    idxs.append(rowoff + xi)
                for idx in idxs:
                    plsc.addupdate_scatter(h2d_v, [idx], ones)

        for k in range(_NCHUNK):
            acc = h2d_v[pl.ds(k * _L, _L)]
            for r in range(1, _NSUB):
                acc = acc + h2d_v[pl.ds(r * _NBINS + k * _L, _L)]
            hist_v[pl.ds(k * _L, _L)] = acc

        pltpu.async_copy(hist_v, shared.at[b, sid], sem_pub)

        for r in range(_NSUB * _NCHUNK):
            h2d_v[pl.ds(r * _L, _L)] = zeros

        @pl.when(jl >= 2)
        def _():
            pltpu.make_async_copy(
                out_b, o_hbm.at[ch, pl.ds(row0, _RPT), :], sem_out).wait()

        pltpu.make_async_copy(hist_v, shared.at[b, sid], sem_pub).wait()
        plsc.subcore_barrier()
        pltpu.sync_copy(shared.at[b], hall_v)
        for k in range(_NCHUNK):
            acc = hall_v[0, pl.ds(k * _L, _L)]
            for r in range(1, _NSUB):
                acc = acc + hall_v[r, pl.ds(k * _L, _L)]
            hist_v[pl.ds(k * _L, _L)] = acc

        acc_cs = zeros
        npix_f = jnp.full((_L,), float(_H * _W), jnp.float32)
        for k in range(_NCHUNK):
            h = hist_v[pl.ds(k * _L, _L)]
            cs = jnp.cumsum(h)
            cs_buf[pl.ds(_L + k * _L, _L)] = cs
            acc_cs = jnp.maximum(acc_cs, jnp.where(cs < npix_f, cs, 0.0))
        last_val = jnp.float32(_H * _W) - jnp.max(acc_cs)

        def _floor_nonneg(v):
            return v.astype(jnp.int32).astype(jnp.float32)

        is_id = last_val > float(_H * _W - 255)

        @pl.when(jnp.logical_not(is_id))
        def _():
            last_vec = jnp.broadcast_to(last_val, (_L,))
            step = _floor_nonneg((npix_f - last_vec) / 255.0)
            half = _floor_nonneg(step * 0.5)
            for k in range(_NCHUNK):
                prev = cs_buf[pl.ds(_L - 1 + k * _L, _L)]
                lv = _floor_nonneg((prev + half) / step)
                lut_v[pl.ds(k * _L, _L)] = jnp.minimum(lv, 255.0) / 255.0

        @pl.when(is_id)
        def _():
            for k in range(_NCHUNK):
                lut_v[pl.ds(k * _L, _L)] = (iota_f + float(k * _L)) / 255.0

        @pl.loop(0, _RPT)
        def _gather(r):
            for k0 in range(0, _NVROW, _U):
                xis = []
                for k in range(k0, k0 + _U):
                    v = in_b[r, pl.ds(k * _L, _L)]
                    xis.append((v * 255.0).astype(jnp.int32))
                outs_u = [plsc.load_gather(lut_v, [xi]) for xi in xis]
                for k in range(k0, k0 + _U):
                    out_b[r, pl.ds(k * _L, _L)] = outs_u[k - k0]

        pltpu.async_copy(out_b, o_hbm.at[ch, pl.ds(row0, _RPT), :], sem_out)

    @pl.loop(0, _CPC, step=2)
    def _channels(j):
        _one_channel(j, 0)
        _one_channel(j + 1, 1)

    for b in range(2):
        pltpu.make_async_copy(
            outs[b], o_hbm.at[ch0 + _CPC - 2 + b, pl.ds(row0, _RPT), :],
            sem_out).wait()


@jax.jit
def kernel(x):
    xf = x.reshape(_NCH, _H, _W)
    cp = pltpu.CompilerParams(use_tc_tiling_on_sc=True)
    if "needs_layout_passes" in pltpu.CompilerParams.__dataclass_fields__:
        cp = dataclasses.replace(cp, needs_layout_passes=False)
    run = pl.kernel(
        _he_kernel,
        out_type=jax.ShapeDtypeStruct((_NCH, _H, _W), jnp.float32),
        mesh=plsc.VectorSubcoreMesh(core_axis_name="c", subcore_axis_name="s"),
        scratch_types=[
            pltpu.VMEM((_RPT, _W), jnp.float32),
            pltpu.VMEM((_RPT, _W), jnp.float32),
            pltpu.VMEM((_RPT, _W), jnp.float32),
            pltpu.VMEM((_RPT, _W), jnp.float32),
            pltpu.VMEM((_NSUB * _NBINS,), jnp.float32),
            pltpu.VMEM((_NBINS,), jnp.float32),
            pltpu.VMEM((_NSUB, _NBINS), jnp.float32),
            pltpu.VMEM((_NBINS,), jnp.float32),
            pltpu.VMEM((_NBINS + _L,), jnp.float32),
            pltpu.VMEM_SHARED((2, _NSUB, _NBINS), jnp.float32),
            pltpu.SemaphoreType.DMA,
            pltpu.SemaphoreType.DMA,
            pltpu.SemaphoreType.DMA,
        ],
        compiler_params=cp,
    )
    return run(xf).reshape(x.shape)


# speedup vs baseline: 704.8418x; 1.0004x over previous
import dataclasses
import functools

import jax
import jax.numpy as jnp
from jax import lax
from jax.experimental import pallas as pl
from jax.experimental.pallas import tpu as pltpu
from jax.experimental.pallas import tpu_sc as plsc

_L = 16
_NSUB = 16
_NCORE = 2
_H = 512
_W = 512
_RPT = _H // _NSUB
_NVROW = _W // _L
_NBINS = 256
_NCHUNK = _NBINS // _L
_NCH = 48
_CPC = _NCH // _NCORE
_U = 16


def _he_kernel(x_hbm, o_hbm, in0, in1, out0, out1, h2d_v, hist_v, hall_v,
               lut_v, cs_buf, shared, sem_in, sem_out, sem_pub):
    cid = lax.axis_index("c")
    sid = lax.axis_index("s")
    row0 = sid * _RPT
    ch0 = cid * _CPC
    iota_i = lax.iota(jnp.int32, _L)
    iota_f = iota_i.astype(jnp.float32)
    ones = jnp.full((_L,), 1.0, jnp.float32)
    zeros = jnp.full((_L,), 0.0, jnp.float32)
    rowoff = iota_i * _NBINS
    ins = (in0, in1)
    outs = (out0, out1)

    pltpu.async_copy(x_hbm.at[ch0, pl.ds(row0, _RPT), :], in0, sem_in)

    def _one_channel(jl, b):
        ch = ch0 + jl
        in_b = ins[b]
        out_b = outs[b]

        pltpu.make_async_copy(
            x_hbm.at[ch, pl.ds(row0, _RPT), :], in_b, sem_in).wait()

        @pl.when(jl + 1 < _CPC)
        def _():
            pltpu.async_copy(
                x_hbm.at[ch + 1, pl.ds(row0, _RPT), :], ins[1 - b], sem_in)

        @pl.loop(0, _RPT)
        def _hist(r):
            for k0 in range(0, _NVROW, _U):
                idxs = []
                for k in range(k0, k0 + _U):
                    v = in_b[r, pl.ds(k * _L, _L)]
                    xi = (v * 255.0).astype(jnp.int32)
                    idxs.append(rowoff + xi)
                for idx in idxs:
                    plsc.addupdate_scatter(h2d_v, [idx], ones)

        for k in range(_NCHUNK):
            acc = h2d_v[pl.ds(k * _L, _L)]
            for r in range(1, _NSUB):
                acc = acc + h2d_v[pl.ds(r * _NBINS + k * _L, _L)]
            hist_v[pl.ds(k * _L, _L)] = acc

        pltpu.async_copy(hist_v, shared.at[b, sid], sem_pub)

        for r in range(_NSUB * _NCHUNK):
            h2d_v[pl.ds(r * _L, _L)] = zeros

        @pl.when(jl >= 2)
        def _():
            pltpu.make_async_copy(
                out_b, o_hbm.at[ch, pl.ds(row0, _RPT), :], sem_out).wait()

        pltpu.make_async_copy(hist_v, shared.at[b, sid], sem_pub).wait()
        plsc.subcore_barrier()
        pltpu.sync_copy(shared.at[b], hall_v)
        for k in range(_NCHUNK):
            acc = hall_v[0, pl.ds(k * _L, _L)]
            for r in range(1, _NSUB):
                acc = acc + hall_v[r, pl.ds(k * _L, _L)]
            hist_v[pl.ds(k * _L, _L)] = acc

        acc_cs = zeros
        npix_f = jnp.full((_L,), float(_H * _W), jnp.float32)
        carry = jnp.float32(0.0)
        for k in range(_NCHUNK):
            h = hist_v[pl.ds(k * _L, _L)]
            cs = jnp.cumsum(h)
            incl = cs + jnp.broadcast_to(carry, (_L,))
            cs_buf[pl.ds(k * _L, _L)] = incl - h
            carry = carry + jnp.sum(h)
            acc_cs = jnp.maximum(acc_cs, jnp.where(incl < npix_f, incl, 0.0))
        last_val = jnp.float32(_H * _W) - jnp.max(acc_cs)

        def _floor_nonneg(v):
            return v.astype(jnp.int32).astype(jnp.float32)

        is_id = last_val > float(_H * _W - 255)

        @pl.when(jnp.logical_not(is_id))
        def _():
            last_vec = jnp.broadcast_to(last_val, (_L,))
            step = _floor_nonneg((npix_f - last_vec) / 255.0)
            half = _floor_nonneg(step * 0.5)
            for k in range(_NCHUNK):
                excl = cs_buf[pl.ds(k * _L, _L)]
                lv = _floor_nonneg((excl + half) / step)
                lut_v[pl.ds(k * _L, _L)] = jnp.minimum(lv, 255.0) / 255.0

        @pl.when(is_id)
        def _():
            for k in range(_NCHUNK):
                lut_v[pl.ds(k * _L, _L)] = (iota_f + float(k * _L)) / 255.0

        @pl.loop(0, _RPT)
        def _gather(r):
            for k0 in range(0, _NVROW, _U):
                xis = []
                for k in range(k0, k0 + _U):
                    v = in_b[r, pl.ds(k * _L, _L)]
                    xis.append((v * 255.0).astype(jnp.int32))
                outs_u = [plsc.load_gather(lut_v, [xi]) for xi in xis]
                for k in range(k0, k0 + _U):
                    out_b[r, pl.ds(k * _L, _L)] = outs_u[k - k0]

        pltpu.async_copy(out_b, o_hbm.at[ch, pl.ds(row0, _RPT), :], sem_out)

    @pl.loop(0, _CPC, step=2)
    def _channels(j):
        _one_channel(j, 0)
        _one_channel(j + 1, 1)

    for b in range(2):
        pltpu.make_async_copy(
            outs[b], o_hbm.at[ch0 + _CPC - 2 + b, pl.ds(row0, _RPT), :],
            sem_out).wait()


@jax.jit
def kernel(x):
    xf = x.reshape(_NCH, _H, _W)
    cp = pltpu.CompilerParams(use_tc_tiling_on_sc=True)
    if "needs_layout_passes" in pltpu.CompilerParams.__dataclass_fields__:
        cp = dataclasses.replace(cp, needs_layout_passes=False)
    run = pl.kernel(
        _he_kernel,
        out_type=jax.ShapeDtypeStruct((_NCH, _H, _W), jnp.float32),
        mesh=plsc.VectorSubcoreMesh(core_axis_name="c", subcore_axis_name="s"),
        scratch_types=[
            pltpu.VMEM((_RPT, _W), jnp.float32),
            pltpu.VMEM((_RPT, _W), jnp.float32),
            pltpu.VMEM((_RPT, _W), jnp.float32),
            pltpu.VMEM((_RPT, _W), jnp.float32),
            pltpu.VMEM((_NSUB * _NBINS,), jnp.float32),
            pltpu.VMEM((_NBINS,), jnp.float32),
            pltpu.VMEM((_NSUB, _NBINS), jnp.float32),
            pltpu.VMEM((_NBINS,), jnp.float32),
            pltpu.VMEM((_NBINS,), jnp.float32),
            pltpu.VMEM_SHARED((2, _NSUB, _NBINS), jnp.float32),
            pltpu.SemaphoreType.DMA,
            pltpu.SemaphoreType.DMA,
            pltpu.SemaphoreType.DMA,
        ],
        compiler_params=cp,
    )
    return run(xf).reshape(x.shape)
